# baseline (speedup 1.0000x reference)
import functools
import jax
import jax.numpy as jnp
from jax import lax
from jax.experimental import pallas as pl
from jax.experimental.pallas import tpu as pltpu
from jax.experimental.pallas import tpu_sc as plsc

N = 10000
E = 320000
H = 128
NC, NS = 2, 16
NW = NC * NS
EPC = E // NC
EPW = E // NW
CH = 80
NCHUNK = EPW // CH
ROWS_PW = 624
OWN = 312
NEG = -3.0e38

_mesh = plsc.VectorSubcoreMesh(core_axis_name="c", subcore_axis_name="s")
_sc_params = pltpu.CompilerParams(needs_layout_passes=False)


def _worker_edge_base(cid, sid):
    return cid * EPC + sid * EPW


CB = 128
CAP = EPW + CB
JUNK = 328


def _deg_body(dst_hbm, out_hbm, lists_hbm, cnts_hbm, acc_sh,
              ones_v, dst_v, bbuf_v, cvec_v, cnt_s):
    cid = lax.axis_index("c")
    sid = lax.axis_index("s")
    iota = lax.broadcasted_iota(jnp.int32, (16,), 0)
    lane0 = iota == 0

    def fill0_body(i, _):
        for j in range(8):
            ones_v[i, pl.ds(j * 16, 16)] = jnp.zeros((16,), jnp.float32)
        return _

    lax.fori_loop(0, CH, fill0_body, jnp.int32(0))
    for j in range(7):
        pltpu.sync_copy(ones_v.at[pl.ds(0, 80)], acc_sh.at[pl.ds(sid * ROWS_PW + j * 80, 80)])
    pltpu.sync_copy(ones_v.at[pl.ds(0, 64)], acc_sh.at[pl.ds(sid * ROWS_PW + 560, 64)])

    @pl.when(sid == NS - 1)
    def _():
        pltpu.sync_copy(ones_v.at[pl.ds(0, 16)], acc_sh.at[pl.ds(9984, 16)])

    def fill1_body(i, _):
        for j in range(8):
            ones_v[i, pl.ds(j * 16, 16)] = jnp.ones((16,), jnp.float32)
        return _

    lax.fori_loop(0, CH, fill1_body, jnp.int32(0))

    def cinit_body(o, _):
        cnt_s[o] = jnp.int32(0)
        return _

    lax.fori_loop(0, 32, cinit_body, jnp.int32(0))
    plsc.subcore_barrier()

    base = _worker_edge_base(cid, sid)
    wk = cid * NS + sid

    def chunk_body(i, _):
        pltpu.sync_copy(dst_hbm.at[pl.ds(base + i * CH, CH)], dst_v)
        pltpu.sync_copy(ones_v, acc_sh.at[dst_v], add=True)

        def grp_body(gidx, car):
            d16 = dst_v[pl.ds(gidx * 16, 16)]
            of = (d16.astype(jnp.float32) + 0.5) * (1.0 / OWN)
            o16 = jnp.minimum(of.astype(jnp.int32), 31)
            eid16 = iota + (base + i * CH + gidx * 16)
            pack16 = (eid16 << 9) | (d16 - o16 * OWN)
            for l in range(16):
                o = o16[l]
                pk = pack16[l]
                c = cnt_s[o]
                cm = c & (CB - 1)
                slot = o * CB + cm
                plsc.store_scatter(bbuf_v, [jnp.full((16,), 0, jnp.int32) + slot],
                                   jnp.full((16,), 0, jnp.int32) + pk, mask=lane0)

                @pl.when(cm == CB - 1)
                def _():
                    lst = (wk * 32 + o) * CAP + (c - (CB - 1))
                    pltpu.sync_copy(bbuf_v.at[pl.ds(o * CB, CB)],
                                    lists_hbm.at[pl.ds(pl.multiple_of(lst, 8), CB)])

                cnt_s[o] = c + 1
            return car

        lax.fori_loop(0, CH // 16, grp_body, jnp.int32(0))
        return _

    lax.fori_loop(0, NCHUNK, chunk_body, jnp.int32(0))

    def tail_body(o, car):
        c = cnt_s[o]
        rem = c & (CB - 1)

        @pl.when(rem != 0)
        def _():
            jk = jnp.full((16,), JUNK, jnp.int32)
            for k in range(CB // 16):
                idxs = o * CB + rem + k * 16 + iota
                plsc.store_scatter(bbuf_v, [idxs], jk, mask=idxs < (o + 1) * CB)
            lst = (wk * 32 + o) * CAP + (c - rem)
            pltpu.sync_copy(bbuf_v.at[pl.ds(o * CB, CB)],
                            lists_hbm.at[pl.ds(pl.multiple_of(lst, 8), CB)])

        nch = (c + CB - 1) >> 7
        plsc.store_scatter(cvec_v, [jnp.full((16,), 0, jnp.int32) + o],
                           jnp.full((16,), 0, jnp.int32) + nch, mask=lane0)
        return car

    lax.fori_loop(0, 32, tail_body, jnp.int32(0))
    pltpu.sync_copy(cvec_v, cnts_hbm.at[pl.ds(wk * 32, 32)])
    plsc.subcore_barrier()

    pltpu.sync_copy(acc_sh.at[pl.ds(sid * ROWS_PW, ROWS_PW)],
                    out_hbm.at[cid, pl.ds(sid * ROWS_PW, ROWS_PW)])

    @pl.when(sid == NS - 1)
    def _():
        pltpu.sync_copy(acc_sh.at[pl.ds(9984, 16)], out_hbm.at[cid, pl.ds(9984, 16)])


def _sc_degree(dst):
    f = pl.kernel(
        _deg_body,
        out_type=(
            jax.ShapeDtypeStruct((NC, N, H), jnp.float32),
            jax.ShapeDtypeStruct((NW * 32 * CAP,), jnp.int32),
            jax.ShapeDtypeStruct((NW * 32,), jnp.int32),
        ),
        mesh=_mesh,
        compiler_params=_sc_params,
        scratch_types=[
            pltpu.VMEM_SHARED((N, H), jnp.float32),
            pltpu.VMEM((CH, H), jnp.float32),
            pltpu.VMEM((CH,), jnp.int32),
            pltpu.VMEM((32 * CB,), jnp.int32),
            pltpu.VMEM((32,), jnp.int32),
            pltpu.SMEM((32,), jnp.int32),
        ],
    )
    return f(dst)


def _agg_body(g_hbm, src_hbm, dst2_hbm, out_hbm, acc_sh, srcst_v, dstst_v,
              rows0_v, rows1_v, sem0, sem1):
    cid = lax.axis_index("c")
    sid = lax.axis_index("s")

    def zero_body(i, _):
        for j in range(8):
            rows0_v[i, pl.ds(j * 16, 16)] = jnp.zeros((16,), jnp.float32)
        return _

    lax.fori_loop(0, CH, zero_body, jnp.int32(0))
    for j in range(7):
        pltpu.sync_copy(rows0_v.at[pl.ds(0, 80)], acc_sh.at[pl.ds(sid * ROWS_PW + j * 80, 80)])
    pltpu.sync_copy(rows0_v.at[pl.ds(0, 64)], acc_sh.at[pl.ds(sid * ROWS_PW + 560, 64)])

    @pl.when(sid == NS - 1)
    def _():
        pltpu.sync_copy(rows0_v.at[pl.ds(0, 16)], acc_sh.at[pl.ds(9984, 16)])

    base = _worker_edge_base(cid, sid)
    wk = cid * NS + sid
    pltpu.sync_copy(src_hbm.at[pl.ds(base, EPW)], srcst_v)
    pltpu.sync_copy(dst2_hbm.at[wk], dstst_v)
    plsc.subcore_barrier()

    cp0 = pltpu.async_copy(g_hbm.at[srcst_v.at[pl.ds(0, CH)]], rows0_v, sem0)

    def pair_body(j, car):
        i0 = 2 * j
        cp0 = pltpu.make_async_copy(g_hbm.at[srcst_v.at[pl.ds(i0 * CH, CH)]], rows0_v, sem0)
        cp0.wait()
        pltpu.async_copy(g_hbm.at[srcst_v.at[pl.ds((i0 + 1) * CH, CH)]], rows1_v, sem1)
        pltpu.sync_copy(rows0_v, acc_sh.at[dstst_v.at[i0]], add=True)
        pltpu.make_async_copy(g_hbm.at[srcst_v.at[pl.ds((i0 + 1) * CH, CH)]], rows1_v, sem1).wait()

        @pl.when(i0 + 2 < NCHUNK)
        def _():
            pltpu.async_copy(g_hbm.at[srcst_v.at[pl.ds((i0 + 2) * CH, CH)]], rows0_v, sem0)

        pltpu.sync_copy(rows1_v, acc_sh.at[dstst_v.at[i0 + 1]], add=True)
        return car

    lax.fori_loop(0, NCHUNK // 2, pair_body, jnp.int32(0))
    pltpu.make_async_copy(g_hbm.at[srcst_v.at[pl.ds((NCHUNK - 1) * CH, CH)]], rows0_v, sem0).wait()
    pltpu.sync_copy(rows0_v, acc_sh.at[dstst_v.at[NCHUNK - 1]], add=True)
    plsc.subcore_barrier()

    pltpu.sync_copy(acc_sh.at[pl.ds(sid * ROWS_PW, ROWS_PW)],
                    out_hbm.at[cid, pl.ds(sid * ROWS_PW, ROWS_PW)])

    @pl.when(sid == NS - 1)
    def _():
        pltpu.sync_copy(acc_sh.at[pl.ds(9984, 16)], out_hbm.at[cid, pl.ds(9984, 16)])


def _sc_gcn_agg(g, src, dst2):
    f = pl.kernel(
        _agg_body,
        out_type=jax.ShapeDtypeStruct((NC, N, H), jnp.float32),
        mesh=_mesh,
        compiler_params=_sc_params,
        scratch_types=[
            pltpu.VMEM_SHARED((N, H), jnp.float32),
            pltpu.VMEM((EPW,), jnp.int32),
            pltpu.VMEM((NCHUNK, CH), jnp.int32),
            pltpu.VMEM((CH, H), jnp.float32),
            pltpu.VMEM((CH, H), jnp.float32),
            pltpu.SemaphoreType.DMA,
            pltpu.SemaphoreType.DMA,
        ],
    )
    return f(g, src, dst2)


def _pre_body(p_hbm, q_hbm, src_hbm, dst_hbm, out_hbm, srcst_v, dstst_v,
              pa0, qa0, pa1, qa1, ob, semp0, semq0, semp1, semq1):
    cid = lax.axis_index("c")
    sid = lax.axis_index("s")
    base = _worker_edge_base(cid, sid)
    pltpu.sync_copy(src_hbm.at[pl.ds(base, EPW)], srcst_v)
    pltpu.sync_copy(dst_hbm.at[pl.ds(base, EPW)], dstst_v)

    def gat(i, pa, qa, semp, semq):
        pltpu.async_copy(p_hbm.at[dstst_v.at[pl.ds(i * CH, CH)]], pa, semp)
        pltpu.async_copy(q_hbm.at[srcst_v.at[pl.ds(i * CH, CH)]], qa, semq)

    def wai(i, pa, qa, semp, semq):
        pltpu.make_async_copy(p_hbm.at[dstst_v.at[pl.ds(i * CH, CH)]], pa, semp).wait()
        pltpu.make_async_copy(q_hbm.at[srcst_v.at[pl.ds(i * CH, CH)]], qa, semq).wait()

    def compute_out(i, pa, qa):
        def row_body(r, car):
            for u in range(2):
                for j in range(8):
                    v = pa[2 * r + u, pl.ds(j * 16, 16)] + qa[2 * r + u, pl.ds(j * 16, 16)]
                    ob[2 * r + u, pl.ds(j * 16, 16)] = jnp.maximum(v, 0.0)
            return car

        lax.fori_loop(0, CH // 2, row_body, jnp.int32(0))
        pltpu.sync_copy(ob, out_hbm.at[pl.ds(base + i * CH, CH)])

    gat(0, pa0, qa0, semp0, semq0)

    def pair_body(jj, car):
        i0 = 2 * jj
        wai(i0, pa0, qa0, semp0, semq0)
        gat(i0 + 1, pa1, qa1, semp1, semq1)
        compute_out(i0, pa0, qa0)
        wai(i0 + 1, pa1, qa1, semp1, semq1)

        @pl.when(i0 + 2 < NCHUNK)
        def _():
            gat(i0 + 2, pa0, qa0, semp0, semq0)

        compute_out(i0 + 1, pa1, qa1)
        return car

    lax.fori_loop(0, NCHUNK // 2, pair_body, jnp.int32(0))
    wai(NCHUNK - 1, pa0, qa0, semp0, semq0)
    compute_out(NCHUNK - 1, pa0, qa0)


def _sc_edge_pre(p, q, src, dst):
    f = pl.kernel(
        _pre_body,
        out_type=jax.ShapeDtypeStruct((E, H), jnp.float32),
        mesh=_mesh,
        compiler_params=_sc_params,
        scratch_types=[
            pltpu.VMEM((EPW,), jnp.int32),
            pltpu.VMEM((EPW,), jnp.int32),
            pltpu.VMEM((CH, H), jnp.float32),
            pltpu.VMEM((CH, H), jnp.float32),
            pltpu.VMEM((CH, H), jnp.float32),
            pltpu.VMEM((CH, H), jnp.float32),
            pltpu.VMEM((CH, H), jnp.float32),
            pltpu.SemaphoreType.DMA,
            pltpu.SemaphoreType.DMA,
            pltpu.SemaphoreType.DMA,
            pltpu.SemaphoreType.DMA,
        ],
    )
    return f(p, q, src, dst)


def _smax_body(m_hbm, lists_hbm, cnts_hbm, out_hbm,
               cnts_v, pk_v, idx_v, lds_v, rows_v, acc_v, sem):
    cid = lax.axis_index("c")
    sid = lax.axis_index("s")
    b = sid * NC + cid
    iota = lax.broadcasted_iota(jnp.int32, (16,), 0)

    def init_body(i, _):
        for j in range(8):
            acc_v[i, pl.ds(j * 16, 16)] = jnp.full((16,), NEG, jnp.float32)
        return _

    lax.fori_loop(0, 344, init_body, jnp.int32(0))

    pltpu.sync_copy(cnts_hbm, cnts_v)
    bb16 = (b >> 4) << 4
    lane = b - bb16

    def wk_body(wk, car):
        row16 = cnts_v[pl.ds(pl.multiple_of(wk * 32 + bb16, 8), 16)]
        nch = jnp.sum(jnp.where(iota == lane, row16, 0))
        lb = (wk * 32 + b) * CAP

        def ch_body(k, c2):
            pltpu.sync_copy(lists_hbm.at[pl.ds(pl.multiple_of(lb + k * CB, 8), CB)], pk_v)
            for t in range(CB // 16):
                pk = pk_v[pl.ds(t * 16, 16)]
                idx_v[pl.ds(t * 16, 16)] = pk >> 9
                lds_v[pl.ds(t * 16, 16)] = pk & 511
            pltpu.async_copy(m_hbm.at[idx_v], rows_v, sem).wait()

            def row_body(r, c3):
                ld = lds_v[pl.ds(r, 16)][0]
                for j in range(8):
                    cur = acc_v[ld, pl.ds(j * 16, 16)]
                    acc_v[ld, pl.ds(j * 16, 16)] = jnp.maximum(cur, rows_v[r, pl.ds(j * 16, 16)])
                return c3

            lax.fori_loop(0, CB, row_body, jnp.int32(0))
            return c2

        lax.fori_loop(0, nch, ch_body, jnp.int32(0))
        return car

    lax.fori_loop(0, NW, wk_body, jnp.int32(0))

    pltpu.sync_copy(acc_v.at[pl.ds(0, OWN)], out_hbm.at[pl.ds(b * OWN, OWN)])

    @pl.when(b == NW - 1)
    def _():
        pltpu.sync_copy(acc_v.at[pl.ds(OWN, 16)], out_hbm.at[pl.ds(9984, 16)])


def _sc_segmax(m, lists, cnts):
    f = pl.kernel(
        _smax_body,
        out_type=jax.ShapeDtypeStruct((N, H), jnp.float32),
        mesh=_mesh,
        compiler_params=_sc_params,
        scratch_types=[
            pltpu.VMEM((NW * 32,), jnp.int32),
            pltpu.VMEM((CB,), jnp.int32),
            pltpu.VMEM((CB,), jnp.int32),
            pltpu.VMEM((CB + 16,), jnp.int32),
            pltpu.VMEM((CB, H), jnp.float32),
            pltpu.VMEM((344, H), jnp.float32),
            pltpu.SemaphoreType.DMA,
        ],
    )
    return f(m, lists, cnts)


BN = 2000
GN = N // BN
BE = 4000
GE = E // BE


def _dinv_from_deg(degp):
    deg = degp[0, :, 0:1] + degp[1, :, 0:1] + 1.0
    return lax.rsqrt(jnp.maximum(deg, 1.0))


def _enc_body(x_ref, ew_ref, eb_ref, lg_ref, lb_ref, w1_ref, degp_ref, g_ref, hw_ref):
    h = jnp.dot(x_ref[...], ew_ref[...], preferred_element_type=jnp.float32) + eb_ref[...]
    h = jnp.maximum(h, 0.0)
    mu = jnp.mean(h, axis=-1, keepdims=True)
    var = jnp.mean((h - mu) ** 2, axis=-1, keepdims=True)
    h = lg_ref[...] * (h - mu) * lax.rsqrt(var + 1e-5) + lb_ref[...]
    dinv = _dinv_from_deg(degp_ref[...])
    hw = jnp.dot(h, w1_ref[...], preferred_element_type=jnp.float32)
    hw_ref[...] = hw
    g_ref[...] = dinv * hw


def _tc_encoder(x, enc_W, enc_b, ln_g, ln_b, W1, degp):
    wspec = pl.BlockSpec((H, H), lambda i: (0, 0))
    vspec = pl.BlockSpec((1, H), lambda i: (0, 0))
    nspec = pl.BlockSpec((BN, H), lambda i: (i, 0))
    dspec = pl.BlockSpec((NC, BN, H), lambda i: (0, i, 0))
    return pl.pallas_call(
        _enc_body,
        grid=(GN,),
        in_specs=[nspec, wspec, vspec, vspec, vspec, wspec, dspec],
        out_specs=[nspec, nspec],
        out_shape=[jax.ShapeDtypeStruct((N, H), jnp.float32)] * 2,
    )(x, enc_W, enc_b, ln_g, ln_b, W1, degp)


def _mid_body(aggp_ref, hw_ref, degp_ref, b_ref, wn_ref, g_ref, hwn_ref):
    dinv = _dinv_from_deg(degp_ref[...])
    agg = aggp_ref[0] + aggp_ref[1]
    h = jnp.maximum(dinv * agg + dinv * dinv * hw_ref[...] + b_ref[...], 0.0)
    hwn = jnp.dot(h, wn_ref[...], preferred_element_type=jnp.float32)
    hwn_ref[...] = hwn
    g_ref[...] = dinv * hwn


def _tc_gcn_mid(aggp, hw, degp, b, Wn):
    wspec = pl.BlockSpec((H, H), lambda i: (0, 0))
    vspec = pl.BlockSpec((1, H), lambda i: (0, 0))
    nspec = pl.BlockSpec((BN, H), lambda i: (i, 0))
    aspec = pl.BlockSpec((NC, BN, H), lambda i: (0, i, 0))
    dspec = pl.BlockSpec((NC, BN, H), lambda i: (0, i, 0))
    return pl.pallas_call(
        _mid_body,
        grid=(GN,),
        in_specs=[aspec, nspec, dspec, vspec, wspec],
        out_specs=[nspec, nspec],
        out_shape=[jax.ShapeDtypeStruct((N, H), jnp.float32)] * 2,
    )(aggp, hw, degp, b, Wn)


def _fin_body(aggp_ref, hw_ref, degp_ref, b_ref, ecw1_ref, ecb1_ref, p_ref, q_ref):
    dinv = _dinv_from_deg(degp_ref[...])
    agg = aggp_ref[0] + aggp_ref[1]
    h = jnp.maximum(dinv * agg + dinv * dinv * hw_ref[...] + b_ref[...], 0.0)
    wa = ecw1_ref[0:H, :]
    wb = ecw1_ref[H:2 * H, :]
    p_ref[...] = jnp.dot(h, wa - wb, preferred_element_type=jnp.float32) + ecb1_ref[...]
    q_ref[...] = jnp.dot(h, wb, preferred_element_type=jnp.float32)


def _tc_gcn_fin(aggp, hw, degp, b, ec_W1, ec_b1):
    vspec = pl.BlockSpec((1, H), lambda i: (0, 0))
    nspec = pl.BlockSpec((BN, H), lambda i: (i, 0))
    aspec = pl.BlockSpec((NC, BN, H), lambda i: (0, i, 0))
    dspec = pl.BlockSpec((NC, BN, H), lambda i: (0, i, 0))
    w2spec = pl.BlockSpec((2 * H, H), lambda i: (0, 0))
    return pl.pallas_call(
        _fin_body,
        grid=(GN,),
        in_specs=[aspec, nspec, dspec, vspec, w2spec, vspec],
        out_specs=[nspec, nspec],
        out_shape=[jax.ShapeDtypeStruct((N, H), jnp.float32)] * 2,
    )(aggp, hw, degp, b, ec_W1, ec_b1)


def _msg_body(pre_ref, w2_ref, b2_ref, m_ref):
    m_ref[...] = jnp.dot(pre_ref[...], w2_ref[...], preferred_element_type=jnp.float32) + b2_ref[...]


def _tc_edge_msg(pre, ec_W2, ec_b2):
    espec = pl.BlockSpec((BE, H), lambda i: (i, 0))
    wspec = pl.BlockSpec((H, H), lambda i: (0, 0))
    vspec = pl.BlockSpec((1, H), lambda i: (0, 0))
    return pl.pallas_call(
        _msg_body,
        grid=(GE,),
        in_specs=[espec, wspec, vspec],
        out_specs=espec,
        out_shape=jax.ShapeDtypeStruct((E, H), jnp.float32),
    )(pre, ec_W2, ec_b2)


def _head_body(sm_ref, pw1_ref, pb1_ref, cw1_ref, cb1_ref, wa_ref, wb_ref, bias_ref, y_ref):
    sm = sm_ref[...]
    h = jnp.where(sm > NEG, sm, 0.0)
    t1 = jnp.maximum(jnp.dot(h, pw1_ref[...], preferred_element_type=jnp.float32) + pb1_ref[...], 0.0)
    t2 = jnp.maximum(jnp.dot(h, cw1_ref[...], preferred_element_type=jnp.float32) + cb1_ref[...], 0.0)
    y = (jnp.dot(t1, wa_ref[...], preferred_element_type=jnp.float32)
         + jnp.dot(t2, wb_ref[...], preferred_element_type=jnp.float32) + bias_ref[...])
    col = lax.broadcasted_iota(jnp.int32, y.shape, 1)
    y_ref[...] = jnp.where(col == 2, jax.nn.sigmoid(y), y)


def _tc_heads(sm, ph_W1, ph_b1, ch_W1, ch_b1, wa, wb, bias):
    wspec = pl.BlockSpec((H, H), lambda i: (0, 0))
    vspec = pl.BlockSpec((1, H), lambda i: (0, 0))
    nspec = pl.BlockSpec((BN, H), lambda i: (i, 0))
    return pl.pallas_call(
        _head_body,
        grid=(GN,),
        in_specs=[nspec, wspec, vspec, wspec, vspec, wspec, wspec, vspec],
        out_specs=nspec,
        out_shape=jax.ShapeDtypeStruct((N, H), jnp.float32),
    )(sm, ph_W1, ph_b1, ch_W1, ch_b1, wa, wb, bias)


def kernel(x, edge_index, enc_W, enc_b, ln_g, ln_b, W1, b1, W2, b2, W3, b3,
           ec_W1, ec_b1, ec_W2, ec_b2, ph_W1, ph_b1, ph_W2, ph_b2,
           ch_W1, ch_b1, ch_W2, ch_b2):
    src = edge_index[0]
    dst = edge_index[1]

    dst3 = dst.reshape(NW, NCHUNK, CH)

    degp, lists, cnts = _sc_degree(dst)

    row = lambda v: v.reshape(1, H)
    g, hw = _tc_encoder(x, enc_W, row(enc_b), row(ln_g), row(ln_b), W1, degp)

    aggp = _sc_gcn_agg(g, src, dst3)
    g, hw = _tc_gcn_mid(aggp, hw, degp, row(b1), W2)
    aggp = _sc_gcn_agg(g, src, dst3)
    g, hw = _tc_gcn_mid(aggp, hw, degp, row(b2), W3)
    aggp = _sc_gcn_agg(g, src, dst3)
    p, q = _tc_gcn_fin(aggp, hw, degp, row(b3), ec_W1, ec_b1.reshape(1, H))

    pre = _sc_edge_pre(p, q, src, dst)
    m = _tc_edge_msg(pre, ec_W2, ec_b2.reshape(1, H))
    sm = _sc_segmax(m, lists, cnts)

    zpad = jnp.zeros((H, H - 3), jnp.float32)
    wa = jnp.concatenate([ph_W2, jnp.zeros((H, 1), jnp.float32), zpad], axis=1)
    wb = jnp.concatenate([jnp.zeros((H, 2), jnp.float32), ch_W2, zpad], axis=1)
    bias = jnp.concatenate([ph_b2, ch_b2, jnp.zeros((H - 3,), jnp.float32)]).reshape(1, H)

    y = _tc_heads(sm, ph_W1, ph_b1.reshape(1, H), ch_W1, ch_b1.reshape(1, H), wa, wb, bias)
    return y[:, :3]


# speedup vs baseline: 2.3488x; 2.3488x over previous
import functools
import jax
import jax.numpy as jnp
from jax import lax
from jax.experimental import pallas as pl
from jax.experimental.pallas import tpu as pltpu
from jax.experimental.pallas import tpu_sc as plsc

N = 10000
E = 320000
H = 128
NC, NS = 2, 16
NW = NC * NS
EPC = E // NC
EPW = E // NW
CH = 80
NCHUNK = EPW // CH
ROWS_PW = 624
OWN = 312
NEG = -3.0e38

_mesh = plsc.VectorSubcoreMesh(core_axis_name="c", subcore_axis_name="s")
_sc_params = pltpu.CompilerParams(needs_layout_passes=False)


def _worker_edge_base(cid, sid):
    return cid * EPC + sid * EPW


CB = 128
CAP = EPW + CB
JUNK = 328


def _deg_body(dst_hbm, out_hbm, lists_hbm, cnts_hbm, acc_sh,
              ones_v, dst_v, bbuf_v, cvec_v, cnt_s):
    cid = lax.axis_index("c")
    sid = lax.axis_index("s")
    iota = lax.broadcasted_iota(jnp.int32, (16,), 0)
    lane0 = iota == 0

    def fill0_body(i, _):
        for j in range(8):
            ones_v[i, pl.ds(j * 16, 16)] = jnp.zeros((16,), jnp.float32)
        return _

    lax.fori_loop(0, CH, fill0_body, jnp.int32(0))
    for j in range(7):
        pltpu.sync_copy(ones_v.at[pl.ds(0, 80)], acc_sh.at[pl.ds(sid * ROWS_PW + j * 80, 80)])
    pltpu.sync_copy(ones_v.at[pl.ds(0, 64)], acc_sh.at[pl.ds(sid * ROWS_PW + 560, 64)])

    @pl.when(sid == NS - 1)
    def _():
        pltpu.sync_copy(ones_v.at[pl.ds(0, 16)], acc_sh.at[pl.ds(9984, 16)])

    def fill1_body(i, _):
        for j in range(8):
            ones_v[i, pl.ds(j * 16, 16)] = jnp.ones((16,), jnp.float32)
        return _

    lax.fori_loop(0, CH, fill1_body, jnp.int32(0))

    def cinit_body(o, _):
        cnt_s[o] = jnp.int32(0)
        return _

    lax.fori_loop(0, 32, cinit_body, jnp.int32(0))
    plsc.subcore_barrier()

    base = _worker_edge_base(cid, sid)
    wk = cid * NS + sid

    def chunk_body(i, _):
        pltpu.sync_copy(dst_hbm.at[pl.ds(base + i * CH, CH)], dst_v)
        pltpu.sync_copy(ones_v, acc_sh.at[dst_v], add=True)

        def grp_body(gidx, car):
            d16 = dst_v[pl.ds(gidx * 16, 16)]
            of = (d16.astype(jnp.float32) + 0.5) * (1.0 / OWN)
            o16 = jnp.minimum(of.astype(jnp.int32), 31)
            eid16 = iota + (base + i * CH + gidx * 16)
            pack16 = (eid16 << 9) | (d16 - o16 * OWN)
            for l in range(16):
                o = o16[l]
                pk = pack16[l]
                c = cnt_s[o]
                cm = c & (CB - 1)
                slot = o * CB + cm
                plsc.store_scatter(bbuf_v, [jnp.full((16,), 0, jnp.int32) + slot],
                                   jnp.full((16,), 0, jnp.int32) + pk, mask=lane0)

                @pl.when(cm == CB - 1)
                def _():
                    lst = (wk * 32 + o) * CAP + (c - (CB - 1))
                    pltpu.sync_copy(bbuf_v.at[pl.ds(o * CB, CB)],
                                    lists_hbm.at[pl.ds(pl.multiple_of(lst, 8), CB)])

                cnt_s[o] = c + 1
            return car

        lax.fori_loop(0, CH // 16, grp_body, jnp.int32(0))
        return _

    lax.fori_loop(0, NCHUNK, chunk_body, jnp.int32(0))

    def tail_body(o, car):
        c = cnt_s[o]
        rem = c & (CB - 1)

        @pl.when(rem != 0)
        def _():
            for k in range(CB // 16):
                idxs = o * CB + rem + k * 16 + iota
                jk = (idxs << 9) | JUNK
                plsc.store_scatter(bbuf_v, [idxs], jk, mask=idxs < (o + 1) * CB)
            lst = (wk * 32 + o) * CAP + (c - rem)
            pltpu.sync_copy(bbuf_v.at[pl.ds(o * CB, CB)],
                            lists_hbm.at[pl.ds(pl.multiple_of(lst, 8), CB)])

        nch = (c + CB - 1) >> 7
        plsc.store_scatter(cvec_v, [jnp.full((16,), 0, jnp.int32) + o],
                           jnp.full((16,), 0, jnp.int32) + nch, mask=lane0)
        return car

    lax.fori_loop(0, 32, tail_body, jnp.int32(0))
    pltpu.sync_copy(cvec_v, cnts_hbm.at[pl.ds(wk * 32, 32)])
    plsc.subcore_barrier()

    pltpu.sync_copy(acc_sh.at[pl.ds(sid * ROWS_PW, ROWS_PW)],
                    out_hbm.at[cid, pl.ds(sid * ROWS_PW, ROWS_PW)])

    @pl.when(sid == NS - 1)
    def _():
        pltpu.sync_copy(acc_sh.at[pl.ds(9984, 16)], out_hbm.at[cid, pl.ds(9984, 16)])


def _sc_degree(dst):
    f = pl.kernel(
        _deg_body,
        out_type=(
            jax.ShapeDtypeStruct((NC, N, H), jnp.float32),
            jax.ShapeDtypeStruct((NW * 32 * CAP,), jnp.int32),
            jax.ShapeDtypeStruct((NW * 32,), jnp.int32),
        ),
        mesh=_mesh,
        compiler_params=_sc_params,
        scratch_types=[
            pltpu.VMEM_SHARED((N, H), jnp.float32),
            pltpu.VMEM((CH, H), jnp.float32),
            pltpu.VMEM((CH,), jnp.int32),
            pltpu.VMEM((32 * CB,), jnp.int32),
            pltpu.VMEM((32,), jnp.int32),
            pltpu.SMEM((32,), jnp.int32),
        ],
    )
    return f(dst)


def _agg_body(g_hbm, src_hbm, dst2_hbm, out_hbm, acc_sh, srcst_v, dstst_v,
              rows0_v, rows1_v, sem0, sem1):
    cid = lax.axis_index("c")
    sid = lax.axis_index("s")

    def zero_body(i, _):
        for j in range(8):
            rows0_v[i, pl.ds(j * 16, 16)] = jnp.zeros((16,), jnp.float32)
        return _

    lax.fori_loop(0, CH, zero_body, jnp.int32(0))
    for j in range(7):
        pltpu.sync_copy(rows0_v.at[pl.ds(0, 80)], acc_sh.at[pl.ds(sid * ROWS_PW + j * 80, 80)])
    pltpu.sync_copy(rows0_v.at[pl.ds(0, 64)], acc_sh.at[pl.ds(sid * ROWS_PW + 560, 64)])

    @pl.when(sid == NS - 1)
    def _():
        pltpu.sync_copy(rows0_v.at[pl.ds(0, 16)], acc_sh.at[pl.ds(9984, 16)])

    base = _worker_edge_base(cid, sid)
    wk = cid * NS + sid
    pltpu.sync_copy(src_hbm.at[pl.ds(base, EPW)], srcst_v)
    pltpu.sync_copy(dst2_hbm.at[wk], dstst_v)
    plsc.subcore_barrier()

    cp0 = pltpu.async_copy(g_hbm.at[srcst_v.at[pl.ds(0, CH)]], rows0_v, sem0)

    def pair_body(j, car):
        i0 = 2 * j
        cp0 = pltpu.make_async_copy(g_hbm.at[srcst_v.at[pl.ds(i0 * CH, CH)]], rows0_v, sem0)
        cp0.wait()
        pltpu.async_copy(g_hbm.at[srcst_v.at[pl.ds((i0 + 1) * CH, CH)]], rows1_v, sem1)
        pltpu.sync_copy(rows0_v, acc_sh.at[dstst_v.at[i0]], add=True)
        pltpu.make_async_copy(g_hbm.at[srcst_v.at[pl.ds((i0 + 1) * CH, CH)]], rows1_v, sem1).wait()

        @pl.when(i0 + 2 < NCHUNK)
        def _():
            pltpu.async_copy(g_hbm.at[srcst_v.at[pl.ds((i0 + 2) * CH, CH)]], rows0_v, sem0)

        pltpu.sync_copy(rows1_v, acc_sh.at[dstst_v.at[i0 + 1]], add=True)
        return car

    lax.fori_loop(0, NCHUNK // 2, pair_body, jnp.int32(0))
    pltpu.make_async_copy(g_hbm.at[srcst_v.at[pl.ds((NCHUNK - 1) * CH, CH)]], rows0_v, sem0).wait()
    pltpu.sync_copy(rows0_v, acc_sh.at[dstst_v.at[NCHUNK - 1]], add=True)
    plsc.subcore_barrier()

    pltpu.sync_copy(acc_sh.at[pl.ds(sid * ROWS_PW, ROWS_PW)],
                    out_hbm.at[cid, pl.ds(sid * ROWS_PW, ROWS_PW)])

    @pl.when(sid == NS - 1)
    def _():
        pltpu.sync_copy(acc_sh.at[pl.ds(9984, 16)], out_hbm.at[cid, pl.ds(9984, 16)])


def _sc_gcn_agg(g, src, dst2):
    f = pl.kernel(
        _agg_body,
        out_type=jax.ShapeDtypeStruct((NC, N, H), jnp.float32),
        mesh=_mesh,
        compiler_params=_sc_params,
        scratch_types=[
            pltpu.VMEM_SHARED((N, H), jnp.float32),
            pltpu.VMEM((EPW,), jnp.int32),
            pltpu.VMEM((NCHUNK, CH), jnp.int32),
            pltpu.VMEM((CH, H), jnp.float32),
            pltpu.VMEM((CH, H), jnp.float32),
            pltpu.SemaphoreType.DMA,
            pltpu.SemaphoreType.DMA,
        ],
    )
    return f(g, src, dst2)


def _pre_body(p_hbm, q_hbm, src_hbm, dst_hbm, out_hbm, srcst_v, dstst_v,
              pa0, qa0, pa1, qa1, ob, semp0, semq0, semp1, semq1):
    cid = lax.axis_index("c")
    sid = lax.axis_index("s")
    base = _worker_edge_base(cid, sid)
    pltpu.sync_copy(src_hbm.at[pl.ds(base, EPW)], srcst_v)
    pltpu.sync_copy(dst_hbm.at[pl.ds(base, EPW)], dstst_v)

    def gat(i, pa, qa, semp, semq):
        pltpu.async_copy(p_hbm.at[dstst_v.at[pl.ds(i * CH, CH)]], pa, semp)
        pltpu.async_copy(q_hbm.at[srcst_v.at[pl.ds(i * CH, CH)]], qa, semq)

    def wai(i, pa, qa, semp, semq):
        pltpu.make_async_copy(p_hbm.at[dstst_v.at[pl.ds(i * CH, CH)]], pa, semp).wait()
        pltpu.make_async_copy(q_hbm.at[srcst_v.at[pl.ds(i * CH, CH)]], qa, semq).wait()

    def compute_out(i, pa, qa):
        def row_body(r, car):
            for u in range(2):
                for j in range(8):
                    v = pa[2 * r + u, pl.ds(j * 16, 16)] + qa[2 * r + u, pl.ds(j * 16, 16)]
                    ob[2 * r + u, pl.ds(j * 16, 16)] = jnp.maximum(v, 0.0)
            return car

        lax.fori_loop(0, CH // 2, row_body, jnp.int32(0))
        pltpu.sync_copy(ob, out_hbm.at[pl.ds(base + i * CH, CH)])

    gat(0, pa0, qa0, semp0, semq0)

    def pair_body(jj, car):
        i0 = 2 * jj
        wai(i0, pa0, qa0, semp0, semq0)
        gat(i0 + 1, pa1, qa1, semp1, semq1)
        compute_out(i0, pa0, qa0)
        wai(i0 + 1, pa1, qa1, semp1, semq1)

        @pl.when(i0 + 2 < NCHUNK)
        def _():
            gat(i0 + 2, pa0, qa0, semp0, semq0)

        compute_out(i0 + 1, pa1, qa1)
        return car

    lax.fori_loop(0, NCHUNK // 2, pair_body, jnp.int32(0))
    wai(NCHUNK - 1, pa0, qa0, semp0, semq0)
    compute_out(NCHUNK - 1, pa0, qa0)


def _sc_edge_pre(p, q, src, dst):
    f = pl.kernel(
        _pre_body,
        out_type=jax.ShapeDtypeStruct((E, H), jnp.float32),
        mesh=_mesh,
        compiler_params=_sc_params,
        scratch_types=[
            pltpu.VMEM((EPW,), jnp.int32),
            pltpu.VMEM((EPW,), jnp.int32),
            pltpu.VMEM((CH, H), jnp.float32),
            pltpu.VMEM((CH, H), jnp.float32),
            pltpu.VMEM((CH, H), jnp.float32),
            pltpu.VMEM((CH, H), jnp.float32),
            pltpu.VMEM((CH, H), jnp.float32),
            pltpu.SemaphoreType.DMA,
            pltpu.SemaphoreType.DMA,
            pltpu.SemaphoreType.DMA,
            pltpu.SemaphoreType.DMA,
        ],
    )
    return f(p, q, src, dst)


def _smax_body(m_hbm, lists_hbm, cnts_hbm, out_hbm,
               cnts_v, pk_v, idx_v, lds_v, rows_v, acc_v, sem):
    cid = lax.axis_index("c")
    sid = lax.axis_index("s")
    b = sid * NC + cid
    iota = lax.broadcasted_iota(jnp.int32, (16,), 0)

    def init_body(i, _):
        for j in range(8):
            acc_v[i, pl.ds(j * 16, 16)] = jnp.full((16,), NEG, jnp.float32)
        return _

    lax.fori_loop(0, 344, init_body, jnp.int32(0))

    pltpu.sync_copy(cnts_hbm, cnts_v)
    bb16 = (b >> 4) << 4
    lane = b - bb16

    def wk_body(wk, car):
        row16 = cnts_v[pl.ds(pl.multiple_of(wk * 32 + bb16, 8), 16)]
        nch = jnp.sum(jnp.where(iota == lane, row16, 0))
        lb = (wk * 32 + b) * CAP

        def ch_body(k, c2):
            pltpu.sync_copy(lists_hbm.at[pl.ds(pl.multiple_of(lb + k * CB, 8), CB)], pk_v)
            for t in range(CB // 16):
                pk = pk_v[pl.ds(t * 16, 16)]
                idx_v[pl.ds(t * 16, 16)] = pk >> 9
                lds_v[pl.ds(t * 16, 16)] = pk & 511
            pltpu.async_copy(m_hbm.at[idx_v], rows_v, sem).wait()

            def t_body(t, c3):
                ld16 = lds_v[pl.ds(t * 16, 16)]
                for rr in range(16):
                    ld = ld16[rr]
                    r = t * 16 + rr
                    for j in range(8):
                        cur = acc_v[ld, pl.ds(j * 16, 16)]
                        acc_v[ld, pl.ds(j * 16, 16)] = jnp.maximum(cur, rows_v[r, pl.ds(j * 16, 16)])
                return c3

            lax.fori_loop(0, CB // 16, t_body, jnp.int32(0))
            return c2

        lax.fori_loop(0, nch, ch_body, jnp.int32(0))
        return car

    lax.fori_loop(0, NW, wk_body, jnp.int32(0))

    pltpu.sync_copy(acc_v.at[pl.ds(0, OWN)], out_hbm.at[pl.ds(b * OWN, OWN)])

    @pl.when(b == NW - 1)
    def _():
        pltpu.sync_copy(acc_v.at[pl.ds(OWN, 16)], out_hbm.at[pl.ds(9984, 16)])


def _sc_segmax(m, lists, cnts):
    f = pl.kernel(
        _smax_body,
        out_type=jax.ShapeDtypeStruct((N, H), jnp.float32),
        mesh=_mesh,
        compiler_params=_sc_params,
        scratch_types=[
            pltpu.VMEM((NW * 32,), jnp.int32),
            pltpu.VMEM((CB,), jnp.int32),
            pltpu.VMEM((CB,), jnp.int32),
            pltpu.VMEM((CB,), jnp.int32),
            pltpu.VMEM((CB, H), jnp.float32),
            pltpu.VMEM((344, H), jnp.float32),
            pltpu.SemaphoreType.DMA,
        ],
    )
    return f(m, lists, cnts)


BN = 2000
GN = N // BN
BE = 4000
GE = E // BE


def _dinv_from_deg(degp):
    deg = degp[0, :, 0:1] + degp[1, :, 0:1] + 1.0
    return lax.rsqrt(jnp.maximum(deg, 1.0))


def _enc_body(x_ref, ew_ref, eb_ref, lg_ref, lb_ref, w1_ref, degp_ref, g_ref, hw_ref):
    h = jnp.dot(x_ref[...], ew_ref[...], preferred_element_type=jnp.float32) + eb_ref[...]
    h = jnp.maximum(h, 0.0)
    mu = jnp.mean(h, axis=-1, keepdims=True)
    var = jnp.mean((h - mu) ** 2, axis=-1, keepdims=True)
    h = lg_ref[...] * (h - mu) * lax.rsqrt(var + 1e-5) + lb_ref[...]
    dinv = _dinv_from_deg(degp_ref[...])
    hw = jnp.dot(h, w1_ref[...], preferred_element_type=jnp.float32)
    hw_ref[...] = hw
    g_ref[...] = dinv * hw


def _tc_encoder(x, enc_W, enc_b, ln_g, ln_b, W1, degp):
    wspec = pl.BlockSpec((H, H), lambda i: (0, 0))
    vspec = pl.BlockSpec((1, H), lambda i: (0, 0))
    nspec = pl.BlockSpec((BN, H), lambda i: (i, 0))
    dspec = pl.BlockSpec((NC, BN, H), lambda i: (0, i, 0))
    return pl.pallas_call(
        _enc_body,
        grid=(GN,),
        in_specs=[nspec, wspec, vspec, vspec, vspec, wspec, dspec],
        out_specs=[nspec, nspec],
        out_shape=[jax.ShapeDtypeStruct((N, H), jnp.float32)] * 2,
    )(x, enc_W, enc_b, ln_g, ln_b, W1, degp)


def _mid_body(aggp_ref, hw_ref, degp_ref, b_ref, wn_ref, g_ref, hwn_ref):
    dinv = _dinv_from_deg(degp_ref[...])
    agg = aggp_ref[0] + aggp_ref[1]
    h = jnp.maximum(dinv * agg + dinv * dinv * hw_ref[...] + b_ref[...], 0.0)
    hwn = jnp.dot(h, wn_ref[...], preferred_element_type=jnp.float32)
    hwn_ref[...] = hwn
    g_ref[...] = dinv * hwn


def _tc_gcn_mid(aggp, hw, degp, b, Wn):
    wspec = pl.BlockSpec((H, H), lambda i: (0, 0))
    vspec = pl.BlockSpec((1, H), lambda i: (0, 0))
    nspec = pl.BlockSpec((BN, H), lambda i: (i, 0))
    aspec = pl.BlockSpec((NC, BN, H), lambda i: (0, i, 0))
    dspec = pl.BlockSpec((NC, BN, H), lambda i: (0, i, 0))
    return pl.pallas_call(
        _mid_body,
        grid=(GN,),
        in_specs=[aspec, nspec, dspec, vspec, wspec],
        out_specs=[nspec, nspec],
        out_shape=[jax.ShapeDtypeStruct((N, H), jnp.float32)] * 2,
    )(aggp, hw, degp, b, Wn)


def _fin_body(aggp_ref, hw_ref, degp_ref, b_ref, ecw1_ref, ecb1_ref, p_ref, q_ref):
    dinv = _dinv_from_deg(degp_ref[...])
    agg = aggp_ref[0] + aggp_ref[1]
    h = jnp.maximum(dinv * agg + dinv * dinv * hw_ref[...] + b_ref[...], 0.0)
    wa = ecw1_ref[0:H, :]
    wb = ecw1_ref[H:2 * H, :]
    p_ref[...] = jnp.dot(h, wa - wb, preferred_element_type=jnp.float32) + ecb1_ref[...]
    q_ref[...] = jnp.dot(h, wb, preferred_element_type=jnp.float32)


def _tc_gcn_fin(aggp, hw, degp, b, ec_W1, ec_b1):
    vspec = pl.BlockSpec((1, H), lambda i: (0, 0))
    nspec = pl.BlockSpec((BN, H), lambda i: (i, 0))
    aspec = pl.BlockSpec((NC, BN, H), lambda i: (0, i, 0))
    dspec = pl.BlockSpec((NC, BN, H), lambda i: (0, i, 0))
    w2spec = pl.BlockSpec((2 * H, H), lambda i: (0, 0))
    return pl.pallas_call(
        _fin_body,
        grid=(GN,),
        in_specs=[aspec, nspec, dspec, vspec, w2spec, vspec],
        out_specs=[nspec, nspec],
        out_shape=[jax.ShapeDtypeStruct((N, H), jnp.float32)] * 2,
    )(aggp, hw, degp, b, ec_W1, ec_b1)


def _msg_body(pre_ref, w2_ref, b2_ref, m_ref):
    m_ref[...] = jnp.dot(pre_ref[...], w2_ref[...], preferred_element_type=jnp.float32) + b2_ref[...]


def _tc_edge_msg(pre, ec_W2, ec_b2):
    espec = pl.BlockSpec((BE, H), lambda i: (i, 0))
    wspec = pl.BlockSpec((H, H), lambda i: (0, 0))
    vspec = pl.BlockSpec((1, H), lambda i: (0, 0))
    return pl.pallas_call(
        _msg_body,
        grid=(GE,),
        in_specs=[espec, wspec, vspec],
        out_specs=espec,
        out_shape=jax.ShapeDtypeStruct((E, H), jnp.float32),
    )(pre, ec_W2, ec_b2)


def _head_body(sm_ref, pw1_ref, pb1_ref, cw1_ref, cb1_ref, wa_ref, wb_ref, bias_ref, y_ref):
    sm = sm_ref[...]
    h = jnp.where(sm > NEG, sm, 0.0)
    t1 = jnp.maximum(jnp.dot(h, pw1_ref[...], preferred_element_type=jnp.float32) + pb1_ref[...], 0.0)
    t2 = jnp.maximum(jnp.dot(h, cw1_ref[...], preferred_element_type=jnp.float32) + cb1_ref[...], 0.0)
    y = (jnp.dot(t1, wa_ref[...], preferred_element_type=jnp.float32)
         + jnp.dot(t2, wb_ref[...], preferred_element_type=jnp.float32) + bias_ref[...])
    col = lax.broadcasted_iota(jnp.int32, y.shape, 1)
    y_ref[...] = jnp.where(col == 2, jax.nn.sigmoid(y), y)


def _tc_heads(sm, ph_W1, ph_b1, ch_W1, ch_b1, wa, wb, bias):
    wspec = pl.BlockSpec((H, H), lambda i: (0, 0))
    vspec = pl.BlockSpec((1, H), lambda i: (0, 0))
    nspec = pl.BlockSpec((BN, H), lambda i: (i, 0))
    return pl.pallas_call(
        _head_body,
        grid=(GN,),
        in_specs=[nspec, wspec, vspec, wspec, vspec, wspec, wspec, vspec],
        out_specs=nspec,
        out_shape=jax.ShapeDtypeStruct((N, H), jnp.float32),
    )(sm, ph_W1, ph_b1, ch_W1, ch_b1, wa, wb, bias)


def kernel(x, edge_index, enc_W, enc_b, ln_g, ln_b, W1, b1, W2, b2, W3, b3,
           ec_W1, ec_b1, ec_W2, ec_b2, ph_W1, ph_b1, ph_W2, ph_b2,
           ch_W1, ch_b1, ch_W2, ch_b2):
    src = edge_index[0]
    dst = edge_index[1]

    dst3 = dst.reshape(NW, NCHUNK, CH)

    degp, lists, cnts = _sc_degree(dst)

    row = lambda v: v.reshape(1, H)
    g, hw = _tc_encoder(x, enc_W, row(enc_b), row(ln_g), row(ln_b), W1, degp)

    aggp = _sc_gcn_agg(g, src, dst3)
    g, hw = _tc_gcn_mid(aggp, hw, degp, row(b1), W2)
    aggp = _sc_gcn_agg(g, src, dst3)
    g, hw = _tc_gcn_mid(aggp, hw, degp, row(b2), W3)
    aggp = _sc_gcn_agg(g, src, dst3)
    p, q = _tc_gcn_fin(aggp, hw, degp, row(b3), ec_W1, ec_b1.reshape(1, H))

    pre = _sc_edge_pre(p, q, src, dst)
    m = _tc_edge_msg(pre, ec_W2, ec_b2.reshape(1, H))
    sm = _sc_segmax(m, lists, cnts)

    zpad = jnp.zeros((H, H - 3), jnp.float32)
    wa = jnp.concatenate([ph_W2, jnp.zeros((H, 1), jnp.float32), zpad], axis=1)
    wb = jnp.concatenate([jnp.zeros((H, 2), jnp.float32), ch_W2, zpad], axis=1)
    bias = jnp.concatenate([ph_b2, ch_b2, jnp.zeros((H - 3,), jnp.float32)]).reshape(1, H)

    y = _tc_heads(sm, ph_W1, ph_b1.reshape(1, H), ch_W1, ch_b1.reshape(1, H), wa, wb, bias)
    return y[:, :3]


# speedup vs baseline: 2.5237x; 1.0745x over previous
import functools
import jax
import jax.numpy as jnp
from jax import lax
from jax.experimental import pallas as pl
from jax.experimental.pallas import tpu as pltpu
from jax.experimental.pallas import tpu_sc as plsc

N = 10000
E = 320000
H = 128
NC, NS = 2, 16
NW = NC * NS
EPC = E // NC
EPW = E // NW
CH = 80
NCHUNK = EPW // CH
ROWS_PW = 624
OWN = 312
NEG = -3.0e38

_mesh = plsc.VectorSubcoreMesh(core_axis_name="c", subcore_axis_name="s")
_sc_params = pltpu.CompilerParams(needs_layout_passes=False)


def _worker_edge_base(cid, sid):
    return cid * EPC + sid * EPW


CB = 128
CAP = EPW + CB
JUNK = 328


def _deg_body(dst_hbm, out_hbm, lists_hbm, cnts_hbm, acc_sh,
              ones_v, dst_v, bbuf_v, cvec_v, cnt_s):
    cid = lax.axis_index("c")
    sid = lax.axis_index("s")
    iota = lax.broadcasted_iota(jnp.int32, (16,), 0)
    lane0 = iota == 0

    def fill0_body(i, _):
        for j in range(8):
            ones_v[i, pl.ds(j * 16, 16)] = jnp.zeros((16,), jnp.float32)
        return _

    lax.fori_loop(0, CH, fill0_body, jnp.int32(0))
    for j in range(7):
        pltpu.sync_copy(ones_v.at[pl.ds(0, 80)], acc_sh.at[pl.ds(sid * ROWS_PW + j * 80, 80)])
    pltpu.sync_copy(ones_v.at[pl.ds(0, 64)], acc_sh.at[pl.ds(sid * ROWS_PW + 560, 64)])

    @pl.when(sid == NS - 1)
    def _():
        pltpu.sync_copy(ones_v.at[pl.ds(0, 16)], acc_sh.at[pl.ds(9984, 16)])

    def fill1_body(i, _):
        for j in range(8):
            ones_v[i, pl.ds(j * 16, 16)] = jnp.ones((16,), jnp.float32)
        return _

    lax.fori_loop(0, CH, fill1_body, jnp.int32(0))

    def cinit_body(o, _):
        cnt_s[o] = jnp.int32(0)
        return _

    lax.fori_loop(0, 32, cinit_body, jnp.int32(0))
    plsc.subcore_barrier()

    base = _worker_edge_base(cid, sid)
    wk = cid * NS + sid

    def chunk_body(i, _):
        pltpu.sync_copy(dst_hbm.at[pl.ds(base + i * CH, CH)], dst_v)
        pltpu.sync_copy(ones_v, acc_sh.at[dst_v], add=True)

        def grp_body(gidx, car):
            d16 = dst_v[pl.ds(gidx * 16, 16)]
            of = (d16.astype(jnp.float32) + 0.5) * (1.0 / OWN)
            o16 = jnp.minimum(of.astype(jnp.int32), 31)
            eid16 = iota + (base + i * CH + gidx * 16)
            pack16 = (eid16 << 9) | (d16 - o16 * OWN)
            for l in range(16):
                o = o16[l]
                pk = pack16[l]
                c = cnt_s[o]
                cm = c & (CB - 1)
                slot = o * CB + cm
                plsc.store_scatter(bbuf_v, [jnp.full((16,), 0, jnp.int32) + slot],
                                   jnp.full((16,), 0, jnp.int32) + pk, mask=lane0)

                @pl.when(cm == CB - 1)
                def _():
                    lst = (wk * 32 + o) * CAP + (c - (CB - 1))
                    pltpu.sync_copy(bbuf_v.at[pl.ds(o * CB, CB)],
                                    lists_hbm.at[pl.ds(pl.multiple_of(lst, 8), CB)])

                cnt_s[o] = c + 1
            return car

        lax.fori_loop(0, CH // 16, grp_body, jnp.int32(0))
        return _

    lax.fori_loop(0, NCHUNK, chunk_body, jnp.int32(0))

    def tail_body(o, car):
        c = cnt_s[o]
        rem = c & (CB - 1)

        @pl.when(rem != 0)
        def _():
            for k in range(CB // 16):
                idxs = o * CB + rem + k * 16 + iota
                jk = (idxs << 9) | JUNK
                plsc.store_scatter(bbuf_v, [idxs], jk, mask=idxs < (o + 1) * CB)
            lst = (wk * 32 + o) * CAP + (c - rem)
            pltpu.sync_copy(bbuf_v.at[pl.ds(o * CB, CB)],
                            lists_hbm.at[pl.ds(pl.multiple_of(lst, 8), CB)])

        nch = (c + CB - 1) >> 7
        plsc.store_scatter(cvec_v, [jnp.full((16,), 0, jnp.int32) + o],
                           jnp.full((16,), 0, jnp.int32) + nch, mask=lane0)
        return car

    lax.fori_loop(0, 32, tail_body, jnp.int32(0))
    pltpu.sync_copy(cvec_v, cnts_hbm.at[pl.ds(wk * 32, 32)])
    plsc.subcore_barrier()

    pltpu.sync_copy(acc_sh.at[pl.ds(sid * ROWS_PW, ROWS_PW)],
                    out_hbm.at[cid, pl.ds(sid * ROWS_PW, ROWS_PW)])

    @pl.when(sid == NS - 1)
    def _():
        pltpu.sync_copy(acc_sh.at[pl.ds(9984, 16)], out_hbm.at[cid, pl.ds(9984, 16)])


def _sc_degree(dst):
    f = pl.kernel(
        _deg_body,
        out_type=(
            jax.ShapeDtypeStruct((NC, N, H), jnp.float32),
            jax.ShapeDtypeStruct((NW * 32 * CAP,), jnp.int32),
            jax.ShapeDtypeStruct((NW * 32,), jnp.int32),
        ),
        mesh=_mesh,
        compiler_params=_sc_params,
        scratch_types=[
            pltpu.VMEM_SHARED((N, H), jnp.float32),
            pltpu.VMEM((CH, H), jnp.float32),
            pltpu.VMEM((CH,), jnp.int32),
            pltpu.VMEM((32 * CB,), jnp.int32),
            pltpu.VMEM((32,), jnp.int32),
            pltpu.SMEM((32,), jnp.int32),
        ],
    )
    return f(dst)


def _agg_body(g_hbm, src_hbm, dst2_hbm, out_hbm, acc_sh, srcst_v, dstst_v,
              rows0_v, rows1_v, sem0, sem1):
    cid = lax.axis_index("c")
    sid = lax.axis_index("s")

    def zero_body(i, _):
        for j in range(8):
            rows0_v[i, pl.ds(j * 16, 16)] = jnp.zeros((16,), jnp.float32)
        return _

    lax.fori_loop(0, CH, zero_body, jnp.int32(0))
    for j in range(7):
        pltpu.sync_copy(rows0_v.at[pl.ds(0, 80)], acc_sh.at[pl.ds(sid * ROWS_PW + j * 80, 80)])
    pltpu.sync_copy(rows0_v.at[pl.ds(0, 64)], acc_sh.at[pl.ds(sid * ROWS_PW + 560, 64)])

    @pl.when(sid == NS - 1)
    def _():
        pltpu.sync_copy(rows0_v.at[pl.ds(0, 16)], acc_sh.at[pl.ds(9984, 16)])

    base = _worker_edge_base(cid, sid)
    wk = cid * NS + sid
    pltpu.sync_copy(src_hbm.at[pl.ds(base, EPW)], srcst_v)
    pltpu.sync_copy(dst2_hbm.at[wk], dstst_v)
    plsc.subcore_barrier()

    cp0 = pltpu.async_copy(g_hbm.at[srcst_v.at[pl.ds(0, CH)]], rows0_v, sem0)

    def pair_body(j, car):
        i0 = 2 * j
        cp0 = pltpu.make_async_copy(g_hbm.at[srcst_v.at[pl.ds(i0 * CH, CH)]], rows0_v, sem0)
        cp0.wait()
        pltpu.async_copy(g_hbm.at[srcst_v.at[pl.ds((i0 + 1) * CH, CH)]], rows1_v, sem1)
        pltpu.sync_copy(rows0_v, acc_sh.at[dstst_v.at[i0]], add=True)
        pltpu.make_async_copy(g_hbm.at[srcst_v.at[pl.ds((i0 + 1) * CH, CH)]], rows1_v, sem1).wait()

        @pl.when(i0 + 2 < NCHUNK)
        def _():
            pltpu.async_copy(g_hbm.at[srcst_v.at[pl.ds((i0 + 2) * CH, CH)]], rows0_v, sem0)

        pltpu.sync_copy(rows1_v, acc_sh.at[dstst_v.at[i0 + 1]], add=True)
        return car

    lax.fori_loop(0, NCHUNK // 2, pair_body, jnp.int32(0))
    pltpu.make_async_copy(g_hbm.at[srcst_v.at[pl.ds((NCHUNK - 1) * CH, CH)]], rows0_v, sem0).wait()
    pltpu.sync_copy(rows0_v, acc_sh.at[dstst_v.at[NCHUNK - 1]], add=True)
    plsc.subcore_barrier()

    pltpu.sync_copy(acc_sh.at[pl.ds(sid * ROWS_PW, ROWS_PW)],
                    out_hbm.at[cid, pl.ds(sid * ROWS_PW, ROWS_PW)])

    @pl.when(sid == NS - 1)
    def _():
        pltpu.sync_copy(acc_sh.at[pl.ds(9984, 16)], out_hbm.at[cid, pl.ds(9984, 16)])


def _sc_gcn_agg(g, src, dst2):
    f = pl.kernel(
        _agg_body,
        out_type=jax.ShapeDtypeStruct((NC, N, H), jnp.float32),
        mesh=_mesh,
        compiler_params=_sc_params,
        scratch_types=[
            pltpu.VMEM_SHARED((N, H), jnp.float32),
            pltpu.VMEM((EPW,), jnp.int32),
            pltpu.VMEM((NCHUNK, CH), jnp.int32),
            pltpu.VMEM((CH, H), jnp.float32),
            pltpu.VMEM((CH, H), jnp.float32),
            pltpu.SemaphoreType.DMA,
            pltpu.SemaphoreType.DMA,
        ],
    )
    return f(g, src, dst2)


def _pre_body(p_hbm, q_hbm, src_hbm, dst_hbm, out_hbm, srcst_v, dstst_v,
              pa0, qa0, pa1, qa1, ob, semp0, semq0, semp1, semq1):
    cid = lax.axis_index("c")
    sid = lax.axis_index("s")
    base = _worker_edge_base(cid, sid)
    pltpu.sync_copy(src_hbm.at[pl.ds(base, EPW)], srcst_v)
    pltpu.sync_copy(dst_hbm.at[pl.ds(base, EPW)], dstst_v)

    def gat(i, pa, qa, semp, semq):
        pltpu.async_copy(p_hbm.at[dstst_v.at[pl.ds(i * CH, CH)]], pa, semp)
        pltpu.async_copy(q_hbm.at[srcst_v.at[pl.ds(i * CH, CH)]], qa, semq)

    def wai(i, pa, qa, semp, semq):
        pltpu.make_async_copy(p_hbm.at[dstst_v.at[pl.ds(i * CH, CH)]], pa, semp).wait()
        pltpu.make_async_copy(q_hbm.at[srcst_v.at[pl.ds(i * CH, CH)]], qa, semq).wait()

    def compute_out(i, pa, qa):
        def row_body(r, car):
            for u in range(2):
                for j in range(8):
                    v = pa[2 * r + u, pl.ds(j * 16, 16)] + qa[2 * r + u, pl.ds(j * 16, 16)]
                    ob[2 * r + u, pl.ds(j * 16, 16)] = jnp.maximum(v, 0.0)
            return car

        lax.fori_loop(0, CH // 2, row_body, jnp.int32(0))
        pltpu.sync_copy(ob, out_hbm.at[pl.ds(base + i * CH, CH)])

    gat(0, pa0, qa0, semp0, semq0)

    def pair_body(jj, car):
        i0 = 2 * jj
        wai(i0, pa0, qa0, semp0, semq0)
        gat(i0 + 1, pa1, qa1, semp1, semq1)
        compute_out(i0, pa0, qa0)
        wai(i0 + 1, pa1, qa1, semp1, semq1)

        @pl.when(i0 + 2 < NCHUNK)
        def _():
            gat(i0 + 2, pa0, qa0, semp0, semq0)

        compute_out(i0 + 1, pa1, qa1)
        return car

    lax.fori_loop(0, NCHUNK // 2, pair_body, jnp.int32(0))
    wai(NCHUNK - 1, pa0, qa0, semp0, semq0)
    compute_out(NCHUNK - 1, pa0, qa0)


def _sc_edge_pre(p, q, src, dst):
    f = pl.kernel(
        _pre_body,
        out_type=jax.ShapeDtypeStruct((E, H), jnp.float32),
        mesh=_mesh,
        compiler_params=_sc_params,
        scratch_types=[
            pltpu.VMEM((EPW,), jnp.int32),
            pltpu.VMEM((EPW,), jnp.int32),
            pltpu.VMEM((CH, H), jnp.float32),
            pltpu.VMEM((CH, H), jnp.float32),
            pltpu.VMEM((CH, H), jnp.float32),
            pltpu.VMEM((CH, H), jnp.float32),
            pltpu.VMEM((CH, H), jnp.float32),
            pltpu.SemaphoreType.DMA,
            pltpu.SemaphoreType.DMA,
            pltpu.SemaphoreType.DMA,
            pltpu.SemaphoreType.DMA,
        ],
    )
    return f(p, q, src, dst)


MAXCHK = 32 * ((EPW + CB - 1) // CB) + 16


def _smax_body(m_hbm, lists_hbm, cnts_hbm, out_hbm,
               cnts_v, lb_v, pk0_v, pk1_v, idx0_v, idx1_v, lds0_v, lds1_v,
               rows0_v, rows1_v, acc_v, sem0, sem1):
    cid = lax.axis_index("c")
    sid = lax.axis_index("s")
    b = sid * NC + cid
    iota = lax.broadcasted_iota(jnp.int32, (16,), 0)
    lane0 = iota == 0

    def init_body(i, _):
        for j in range(8):
            acc_v[i, pl.ds(j * 16, 16)] = jnp.full((16,), NEG, jnp.float32)
        return _

    lax.fori_loop(0, 344, init_body, jnp.int32(0))

    pltpu.sync_copy(cnts_hbm, cnts_v)
    bb16 = (b >> 4) << 4
    lane = b - bb16

    def wk_body(wk, t):
        row16 = cnts_v[pl.ds(pl.multiple_of(wk * 32 + bb16, 8), 16)]
        nch = jnp.sum(jnp.where(iota == lane, row16, 0))
        lbase = (wk * 32 + b) * CAP

        def kb(k, t2):
            plsc.store_scatter(lb_v, [jnp.full((16,), 0, jnp.int32) + t2],
                               jnp.full((16,), 0, jnp.int32) + (lbase + k * CB), mask=lane0)
            return t2 + 1

        return lax.fori_loop(0, nch, kb, t)

    tot = lax.fori_loop(0, NW, wk_body, jnp.int32(0))

    def lbat(t):
        return lb_v[pl.ds(t, 16)][0]

    def load_idx(t, pk_v, idx_v, lds_v):
        pltpu.sync_copy(lists_hbm.at[pl.ds(pl.multiple_of(lbat(t), 8), CB)], pk_v)
        for tt in range(CB // 16):
            pk = pk_v[pl.ds(tt * 16, 16)]
            idx_v[pl.ds(tt * 16, 16)] = pk >> 9
            lds_v[pl.ds(tt * 16, 16)] = pk & 511

    def rmw(lds_v, rows_v):
        def t_body(t, c3):
            ld16 = lds_v[pl.ds(t * 16, 16)]
            for rr in range(16):
                ld = ld16[rr]
                r = t * 16 + rr
                for j in range(8):
                    cur = acc_v[ld, pl.ds(j * 16, 16)]
                    acc_v[ld, pl.ds(j * 16, 16)] = jnp.maximum(cur, rows_v[r, pl.ds(j * 16, 16)])
            return c3

        lax.fori_loop(0, CB // 16, t_body, jnp.int32(0))

    @pl.when(tot > 0)
    def _():
        load_idx(0, pk0_v, idx0_v, lds0_v)
        pltpu.async_copy(m_hbm.at[idx0_v], rows0_v, sem0)

    def pair_body(jj, car):
        t0 = 2 * jj
        pltpu.make_async_copy(m_hbm.at[idx0_v], rows0_v, sem0).wait()

        @pl.when(t0 + 1 < tot)
        def _():
            load_idx(t0 + 1, pk1_v, idx1_v, lds1_v)
            pltpu.async_copy(m_hbm.at[idx1_v], rows1_v, sem1)

        rmw(lds0_v, rows0_v)

        @pl.when(t0 + 1 < tot)
        def _():
            pltpu.make_async_copy(m_hbm.at[idx1_v], rows1_v, sem1).wait()

            @pl.when(t0 + 2 < tot)
            def _():
                load_idx(t0 + 2, pk0_v, idx0_v, lds0_v)
                pltpu.async_copy(m_hbm.at[idx0_v], rows0_v, sem0)

            rmw(lds1_v, rows1_v)

        return car

    lax.fori_loop(0, (tot + 1) // 2, pair_body, jnp.int32(0))

    pltpu.sync_copy(acc_v.at[pl.ds(0, OWN)], out_hbm.at[pl.ds(b * OWN, OWN)])

    @pl.when(b == NW - 1)
    def _():
        pltpu.sync_copy(acc_v.at[pl.ds(OWN, 16)], out_hbm.at[pl.ds(9984, 16)])


def _sc_segmax(m, lists, cnts):
    f = pl.kernel(
        _smax_body,
        out_type=jax.ShapeDtypeStruct((N, H), jnp.float32),
        mesh=_mesh,
        compiler_params=_sc_params,
        scratch_types=[
            pltpu.VMEM((NW * 32,), jnp.int32),
            pltpu.VMEM((MAXCHK,), jnp.int32),
            pltpu.VMEM((CB,), jnp.int32),
            pltpu.VMEM((CB,), jnp.int32),
            pltpu.VMEM((CB,), jnp.int32),
            pltpu.VMEM((CB,), jnp.int32),
            pltpu.VMEM((CB,), jnp.int32),
            pltpu.VMEM((CB,), jnp.int32),
            pltpu.VMEM((CB, H), jnp.float32),
            pltpu.VMEM((CB, H), jnp.float32),
            pltpu.VMEM((344, H), jnp.float32),
            pltpu.SemaphoreType.DMA,
            pltpu.SemaphoreType.DMA,
        ],
    )
    return f(m, lists, cnts)


BN = 2000
GN = N // BN
BE = 4000
GE = E // BE


def _dinv_from_deg(degp):
    deg = degp[0, :, 0:1] + degp[1, :, 0:1] + 1.0
    return lax.rsqrt(jnp.maximum(deg, 1.0))


def _enc_body(x_ref, ew_ref, eb_ref, lg_ref, lb_ref, w1_ref, degp_ref, g_ref, hw_ref):
    h = jnp.dot(x_ref[...], ew_ref[...], preferred_element_type=jnp.float32) + eb_ref[...]
    h = jnp.maximum(h, 0.0)
    mu = jnp.mean(h, axis=-1, keepdims=True)
    var = jnp.mean((h - mu) ** 2, axis=-1, keepdims=True)
    h = lg_ref[...] * (h - mu) * lax.rsqrt(var + 1e-5) + lb_ref[...]
    dinv = _dinv_from_deg(degp_ref[...])
    hw = jnp.dot(h, w1_ref[...], preferred_element_type=jnp.float32)
    hw_ref[...] = hw
    g_ref[...] = dinv * hw


def _tc_encoder(x, enc_W, enc_b, ln_g, ln_b, W1, degp):
    wspec = pl.BlockSpec((H, H), lambda i: (0, 0))
    vspec = pl.BlockSpec((1, H), lambda i: (0, 0))
    nspec = pl.BlockSpec((BN, H), lambda i: (i, 0))
    dspec = pl.BlockSpec((NC, BN, H), lambda i: (0, i, 0))
    return pl.pallas_call(
        _enc_body,
        grid=(GN,),
        in_specs=[nspec, wspec, vspec, vspec, vspec, wspec, dspec],
        out_specs=[nspec, nspec],
        out_shape=[jax.ShapeDtypeStruct((N, H), jnp.float32)] * 2,
    )(x, enc_W, enc_b, ln_g, ln_b, W1, degp)


def _mid_body(aggp_ref, hw_ref, degp_ref, b_ref, wn_ref, g_ref, hwn_ref):
    dinv = _dinv_from_deg(degp_ref[...])
    agg = aggp_ref[0] + aggp_ref[1]
    h = jnp.maximum(dinv * agg + dinv * dinv * hw_ref[...] + b_ref[...], 0.0)
    hwn = jnp.dot(h, wn_ref[...], preferred_element_type=jnp.float32)
    hwn_ref[...] = hwn
    g_ref[...] = dinv * hwn


def _tc_gcn_mid(aggp, hw, degp, b, Wn):
    wspec = pl.BlockSpec((H, H), lambda i: (0, 0))
    vspec = pl.BlockSpec((1, H), lambda i: (0, 0))
    nspec = pl.BlockSpec((BN, H), lambda i: (i, 0))
    aspec = pl.BlockSpec((NC, BN, H), lambda i: (0, i, 0))
    dspec = pl.BlockSpec((NC, BN, H), lambda i: (0, i, 0))
    return pl.pallas_call(
        _mid_body,
        grid=(GN,),
        in_specs=[aspec, nspec, dspec, vspec, wspec],
        out_specs=[nspec, nspec],
        out_shape=[jax.ShapeDtypeStruct((N, H), jnp.float32)] * 2,
    )(aggp, hw, degp, b, Wn)


def _fin_body(aggp_ref, hw_ref, degp_ref, b_ref, ecw1_ref, ecb1_ref, p_ref, q_ref):
    dinv = _dinv_from_deg(degp_ref[...])
    agg = aggp_ref[0] + aggp_ref[1]
    h = jnp.maximum(dinv * agg + dinv * dinv * hw_ref[...] + b_ref[...], 0.0)
    wa = ecw1_ref[0:H, :]
    wb = ecw1_ref[H:2 * H, :]
    p_ref[...] = jnp.dot(h, wa - wb, preferred_element_type=jnp.float32) + ecb1_ref[...]
    q_ref[...] = jnp.dot(h, wb, preferred_element_type=jnp.float32)


def _tc_gcn_fin(aggp, hw, degp, b, ec_W1, ec_b1):
    vspec = pl.BlockSpec((1, H), lambda i: (0, 0))
    nspec = pl.BlockSpec((BN, H), lambda i: (i, 0))
    aspec = pl.BlockSpec((NC, BN, H), lambda i: (0, i, 0))
    dspec = pl.BlockSpec((NC, BN, H), lambda i: (0, i, 0))
    w2spec = pl.BlockSpec((2 * H, H), lambda i: (0, 0))
    return pl.pallas_call(
        _fin_body,
        grid=(GN,),
        in_specs=[aspec, nspec, dspec, vspec, w2spec, vspec],
        out_specs=[nspec, nspec],
        out_shape=[jax.ShapeDtypeStruct((N, H), jnp.float32)] * 2,
    )(aggp, hw, degp, b, ec_W1, ec_b1)


def _msg_body(pre_ref, w2_ref, b2_ref, m_ref):
    m_ref[...] = jnp.dot(pre_ref[...], w2_ref[...], preferred_element_type=jnp.float32) + b2_ref[...]


def _tc_edge_msg(pre, ec_W2, ec_b2):
    espec = pl.BlockSpec((BE, H), lambda i: (i, 0))
    wspec = pl.BlockSpec((H, H), lambda i: (0, 0))
    vspec = pl.BlockSpec((1, H), lambda i: (0, 0))
    return pl.pallas_call(
        _msg_body,
        grid=(GE,),
        in_specs=[espec, wspec, vspec],
        out_specs=espec,
        out_shape=jax.ShapeDtypeStruct((E, H), jnp.float32),
    )(pre, ec_W2, ec_b2)


def _head_body(sm_ref, pw1_ref, pb1_ref, cw1_ref, cb1_ref, wa_ref, wb_ref, bias_ref, y_ref):
    sm = sm_ref[...]
    h = jnp.where(sm > NEG, sm, 0.0)
    t1 = jnp.maximum(jnp.dot(h, pw1_ref[...], preferred_element_type=jnp.float32) + pb1_ref[...], 0.0)
    t2 = jnp.maximum(jnp.dot(h, cw1_ref[...], preferred_element_type=jnp.float32) + cb1_ref[...], 0.0)
    y = (jnp.dot(t1, wa_ref[...], preferred_element_type=jnp.float32)
         + jnp.dot(t2, wb_ref[...], preferred_element_type=jnp.float32) + bias_ref[...])
    col = lax.broadcasted_iota(jnp.int32, y.shape, 1)
    y_ref[...] = jnp.where(col == 2, jax.nn.sigmoid(y), y)


def _tc_heads(sm, ph_W1, ph_b1, ch_W1, ch_b1, wa, wb, bias):
    wspec = pl.BlockSpec((H, H), lambda i: (0, 0))
    vspec = pl.BlockSpec((1, H), lambda i: (0, 0))
    nspec = pl.BlockSpec((BN, H), lambda i: (i, 0))
    return pl.pallas_call(
        _head_body,
        grid=(GN,),
        in_specs=[nspec, wspec, vspec, wspec, vspec, wspec, wspec, vspec],
        out_specs=nspec,
        out_shape=jax.ShapeDtypeStruct((N, H), jnp.float32),
    )(sm, ph_W1, ph_b1, ch_W1, ch_b1, wa, wb, bias)


def kernel(x, edge_index, enc_W, enc_b, ln_g, ln_b, W1, b1, W2, b2, W3, b3,
           ec_W1, ec_b1, ec_W2, ec_b2, ph_W1, ph_b1, ph_W2, ph_b2,
           ch_W1, ch_b1, ch_W2, ch_b2):
    src = edge_index[0]
    dst = edge_index[1]

    dst3 = dst.reshape(NW, NCHUNK, CH)

    degp, lists, cnts = _sc_degree(dst)

    row = lambda v: v.reshape(1, H)
    g, hw = _tc_encoder(x, enc_W, row(enc_b), row(ln_g), row(ln_b), W1, degp)

    aggp = _sc_gcn_agg(g, src, dst3)
    g, hw = _tc_gcn_mid(aggp, hw, degp, row(b1), W2)
    aggp = _sc_gcn_agg(g, src, dst3)
    g, hw = _tc_gcn_mid(aggp, hw, degp, row(b2), W3)
    aggp = _sc_gcn_agg(g, src, dst3)
    p, q = _tc_gcn_fin(aggp, hw, degp, row(b3), ec_W1, ec_b1.reshape(1, H))

    pre = _sc_edge_pre(p, q, src, dst)
    m = _tc_edge_msg(pre, ec_W2, ec_b2.reshape(1, H))
    sm = _sc_segmax(m, lists, cnts)

    zpad = jnp.zeros((H, H - 3), jnp.float32)
    wa = jnp.concatenate([ph_W2, jnp.zeros((H, 1), jnp.float32), zpad], axis=1)
    wb = jnp.concatenate([jnp.zeros((H, 2), jnp.float32), ch_W2, zpad], axis=1)
    bias = jnp.concatenate([ph_b2, ch_b2, jnp.zeros((H - 3,), jnp.float32)]).reshape(1, H)

    y = _tc_heads(sm, ph_W1, ph_b1.reshape(1, H), ch_W1, ch_b1.reshape(1, H), wa, wb, bias)
    return y[:, :3]
